# Initial kernel scaffold; baseline (speedup 1.0000x reference)
import jax
import jax.numpy as jnp
from jax.experimental import pallas as pl


def kernel(x, atom_to_cycle, emb_table):
    raise NotImplementedError("write your pallas kernel here")


# speedup vs baseline: 28.8995x; 28.8995x over previous
import functools

import jax
import jax.numpy as jnp
from jax import lax
from jax.experimental import pallas as pl
from jax.experimental.pallas import tpu as pltpu
from jax.experimental.pallas import tpu_sc as plsc

N_NODES = 10000
N_PAIRS = 320000
HIDDEN = 128
VOCAB = 22
N_CYCLES = 10000
VPAD = 32
HSIZE = N_CYCLES * VPAD

NC, NS = 2, 16
NW = NC * NS
CHUNK = N_PAIRS // NW
ROWS = CHUNK // 16
HTILE = HSIZE // NS


def _sc_hist_body(a0_hbm, a1_hbm, x_hbm, c_hbm,
                  x_v, a0_v, a1_v, idx_v, ones_v, zero_v, hist_sh):
    c = lax.axis_index("c")
    s = lax.axis_index("s")
    w = c * NS + s
    base = w * CHUNK

    pltpu.sync_copy(x_hbm, x_v)
    pltpu.sync_copy(a0_hbm.at[pl.ds(base, CHUNK)], a0_v)
    pltpu.sync_copy(a1_hbm.at[pl.ds(base, CHUNK)], a1_v)

    def zloop(i, carry):
        zero_v[pl.ds(i * 16, 16)] = jnp.zeros((16,), jnp.float32)
        return carry
    lax.fori_loop(0, HTILE // 16, zloop, 0)
    pltpu.sync_copy(zero_v, hist_sh.at[pl.ds(s * HTILE, HTILE)])

    ones16 = jnp.ones((16,), jnp.float32)
    def iloop(i, carry):
        a0_16 = a0_v[pl.ds(i * 16, 16)]
        v16 = plsc.load_gather(x_v, [a0_16])
        a1_16 = a1_v[pl.ds(i * 16, 16)]
        idx_v[pl.ds(i * 16, 16)] = (a1_16 << 5) | v16
        ones_v[pl.ds(i * 16, 16)] = ones16
        return carry
    lax.fori_loop(0, ROWS, iloop, 0)

    plsc.subcore_barrier()
    pltpu.sync_copy(ones_v, hist_sh.at[idx_v], add=True)
    plsc.subcore_barrier()

    pltpu.sync_copy(hist_sh.at[pl.ds(s * HTILE, HTILE)], zero_v)
    pltpu.sync_copy(zero_v, c_hbm.at[pl.ds(c * HSIZE + s * HTILE, HTILE)])


_sc_hist = functools.partial(
    pl.kernel,
    out_type=jax.ShapeDtypeStruct((NC * HSIZE,), jnp.float32),
    mesh=plsc.VectorSubcoreMesh(core_axis_name="c", subcore_axis_name="s"),
    scratch_types=[
        pltpu.VMEM((N_NODES,), jnp.int32),
        pltpu.VMEM((CHUNK,), jnp.int32),
        pltpu.VMEM((CHUNK,), jnp.int32),
        pltpu.VMEM((CHUNK,), jnp.int32),
        pltpu.VMEM((CHUNK,), jnp.float32),
        pltpu.VMEM((HTILE,), jnp.float32),
        pltpu.VMEM_SHARED((HSIZE,), jnp.float32),
    ],
    compiler_params=pltpu.CompilerParams(needs_layout_passes=False),
)(_sc_hist_body)


def _tc_mm_body(c_ref, emb_ref, o_ref):
    o_ref[...] = jnp.dot(c_ref[0] + c_ref[1], emb_ref[...],
                         preferred_element_type=jnp.float32)


def kernel(x, atom_to_cycle, emb_table):
    a0 = atom_to_cycle[0]
    a1 = atom_to_cycle[1]
    c = _sc_hist(a0, a1, x)
    c3 = c.reshape(NC, N_CYCLES, VPAD)
    emb_pad = jnp.zeros((VPAD, HIDDEN), emb_table.dtype).at[:VOCAB].set(emb_table)
    out = pl.pallas_call(
        _tc_mm_body,
        out_shape=jax.ShapeDtypeStruct((N_CYCLES, HIDDEN), jnp.float32),
    )(c3, emb_pad)
    return out


# speedup vs baseline: 47.6973x; 1.6505x over previous
import functools

import jax
import jax.numpy as jnp
from jax import lax
from jax.experimental import pallas as pl
from jax.experimental.pallas import tpu as pltpu
from jax.experimental.pallas import tpu_sc as plsc

N_NODES = 10000
N_PAIRS = 320000
HIDDEN = 128
VOCAB = 22
N_CYCLES = 10000

STRIPE = 2560
HSIZE = STRIPE * 128
NC, NS = 2, 16
CHUNK = N_PAIRS // (NC * NS)
ROWS = CHUNK // 16
HTILE = HSIZE // NS
ZCH = 2048


def _sc_hist_body(atc_hbm, x_hbm, c_hbm,
                  x_v, a0_v, a1_v, idx_v, ones_v, zero_v, wb_v, hist_sh,
                  in_sem, z_sem):
    cid = lax.axis_index("c")
    s = lax.axis_index("s")
    w = cid * NS + s
    base = w * CHUNK

    cp_x = pltpu.async_copy(x_hbm, x_v, in_sem)
    cp_a0 = pltpu.async_copy(atc_hbm.at[pl.ds(base, CHUNK)], a0_v, in_sem)
    cp_a1 = pltpu.async_copy(atc_hbm.at[pl.ds(N_PAIRS + base, CHUNK)],
                             a1_v, in_sem)

    def zloop(i, carry):
        zero_v[pl.ds(i * 16, 16)] = jnp.zeros((16,), jnp.float32)
        return carry
    lax.fori_loop(0, ZCH // 16, zloop, 0)
    zcps = [
        pltpu.async_copy(zero_v, hist_sh.at[pl.ds(s * HTILE + j * ZCH, ZCH)],
                         z_sem)
        for j in range(HTILE // ZCH)
    ]

    ones16 = jnp.ones((16,), jnp.float32)
    def oloop(i, carry):
        ones_v[pl.ds(i * 16, 16)] = ones16
        return carry
    lax.fori_loop(0, ROWS, oloop, 0)

    cp_x.wait()
    cp_a0.wait()
    cp_a1.wait()

    def iloop(i, carry):
        a0_16 = a0_v[pl.ds(i * 16, 16)]
        v16 = plsc.load_gather(x_v, [a0_16])
        c16 = a1_v[pl.ds(i * 16, 16)]
        g16 = (c16 * 26215) >> 26
        r16 = c16 - ((g16 << 11) + (g16 << 9))
        idx_v[pl.ds(i * 16, 16)] = (r16 << 7) | (g16 << 5) | v16
        return carry
    lax.fori_loop(0, ROWS, iloop, 0)

    for cp in zcps:
        cp.wait()
    plsc.subcore_barrier()
    pltpu.sync_copy(ones_v, hist_sh.at[idx_v], add=True)
    plsc.subcore_barrier()

    pltpu.sync_copy(hist_sh.at[pl.ds(s * HTILE, HTILE)], wb_v)
    pltpu.sync_copy(wb_v, c_hbm.at[pl.ds(cid * HSIZE + s * HTILE, HTILE)])


_sc_hist = functools.partial(
    pl.kernel,
    out_type=jax.ShapeDtypeStruct((NC * HSIZE,), jnp.float32),
    mesh=plsc.VectorSubcoreMesh(core_axis_name="c", subcore_axis_name="s"),
    scratch_types=[
        pltpu.VMEM((N_NODES,), jnp.int32),
        pltpu.VMEM((CHUNK,), jnp.int32),
        pltpu.VMEM((CHUNK,), jnp.int32),
        pltpu.VMEM((CHUNK,), jnp.int32),
        pltpu.VMEM((CHUNK,), jnp.float32),
        pltpu.VMEM((ZCH,), jnp.float32),
        pltpu.VMEM((HTILE,), jnp.float32),
        pltpu.VMEM_SHARED((HSIZE,), jnp.float32),
        pltpu.SemaphoreType.DMA,
        pltpu.SemaphoreType.DMA,
    ],
    compiler_params=pltpu.CompilerParams(needs_layout_passes=False),
)(_sc_hist_body)


def _tc_mm_body(c0_ref, c1_ref, e_ref, o_ref):
    o_ref[...] = jnp.dot(c0_ref[...] + c1_ref[...], e_ref[0],
                         preferred_element_type=jnp.float32)


def kernel(x, atom_to_cycle, emb_table):
    atc_flat = atom_to_cycle.reshape(-1)
    c = _sc_hist(atc_flat, x)
    c2d = c.reshape(NC * STRIPE, 128)
    e4 = jnp.zeros((4, 128, HIDDEN), emb_table.dtype)
    for g in range(4):
        e4 = e4.at[g, g * 32:g * 32 + VOCAB, :].set(emb_table)
    out = pl.pallas_call(
        _tc_mm_body,
        grid=(4,),
        in_specs=[
            pl.BlockSpec((STRIPE, 128), lambda g: (0, 0)),
            pl.BlockSpec((STRIPE, 128), lambda g: (1, 0)),
            pl.BlockSpec((1, 128, HIDDEN), lambda g: (g, 0, 0)),
        ],
        out_specs=pl.BlockSpec((STRIPE, HIDDEN), lambda g: (g, 0)),
        out_shape=jax.ShapeDtypeStruct((N_CYCLES, HIDDEN), jnp.float32),
    )(c2d, c2d, e4)
    return out


# speedup vs baseline: 48.1080x; 1.0086x over previous
import functools

import jax
import jax.numpy as jnp
from jax import lax
from jax.experimental import pallas as pl
from jax.experimental.pallas import tpu as pltpu
from jax.experimental.pallas import tpu_sc as plsc

N_NODES = 10000
N_PAIRS = 320000
HIDDEN = 128
VOCAB = 22
N_CYCLES = 10000

STRIPE = 2560
HSIZE = STRIPE * 128
NC, NS = 2, 16
CHUNK = N_PAIRS // (NC * NS)
ROWS = CHUNK // 16
HTILE = HSIZE // NS
ZCH = 2048
SLAB = 10240


def _sc_hist_body(atc_hbm, x_hbm, c_hbm,
                  x_v, atc_v, idx_v, ones_v, zero_v, wb_v, hist_sh,
                  in_sem, z_sem):
    cid = lax.axis_index("c")
    s = lax.axis_index("s")
    w = cid * NS + s
    base = w * CHUNK

    start = jnp.minimum(base - base % 128, N_PAIRS - SLAB)
    start = pl.multiple_of(start, 128)
    off = base - start
    cp_x = pltpu.async_copy(x_hbm, x_v, in_sem)
    cp_a = pltpu.async_copy(atc_hbm.at[:, pl.ds(start, SLAB)], atc_v, in_sem)

    def zloop(i, carry):
        zero_v[pl.ds(i * 16, 16)] = jnp.zeros((16,), jnp.float32)
        return carry
    lax.fori_loop(0, ZCH // 16, zloop, 0)
    zcps = [
        pltpu.async_copy(zero_v, hist_sh.at[pl.ds(s * HTILE + j * ZCH, ZCH)],
                         z_sem)
        for j in range(HTILE // ZCH)
    ]

    ones16 = jnp.ones((16,), jnp.float32)
    def oloop(i, carry):
        ones_v[pl.ds(i * 16, 16)] = ones16
        return carry
    lax.fori_loop(0, ROWS, oloop, 0)

    cp_x.wait()
    cp_a.wait()

    def iloop(i, carry):
        a0_16 = atc_v[0, pl.ds(off + i * 16, 16)]
        v16 = plsc.load_gather(x_v, [a0_16])
        c16 = atc_v[1, pl.ds(off + i * 16, 16)]
        g16 = (c16 * 26215) >> 26
        r16 = c16 - ((g16 << 11) + (g16 << 9))
        idx_v[pl.ds(i * 16, 16)] = (r16 << 7) | (g16 << 5) | v16
        return carry
    lax.fori_loop(0, ROWS, iloop, 0)

    for cp in zcps:
        cp.wait()
    plsc.subcore_barrier()
    pltpu.sync_copy(ones_v, hist_sh.at[idx_v], add=True)
    plsc.subcore_barrier()

    pltpu.sync_copy(hist_sh.at[pl.ds(s * HTILE, HTILE)], wb_v)
    pltpu.sync_copy(wb_v, c_hbm.at[pl.ds(cid * HSIZE + s * HTILE, HTILE)])


_sc_hist = functools.partial(
    pl.kernel,
    out_type=jax.ShapeDtypeStruct((NC * HSIZE,), jnp.float32),
    mesh=plsc.VectorSubcoreMesh(core_axis_name="c", subcore_axis_name="s"),
    scratch_types=[
        pltpu.VMEM((N_NODES,), jnp.int32),
        pltpu.VMEM((2, SLAB), jnp.int32),
        pltpu.VMEM((CHUNK,), jnp.int32),
        pltpu.VMEM((CHUNK,), jnp.float32),
        pltpu.VMEM((ZCH,), jnp.float32),
        pltpu.VMEM((HTILE,), jnp.float32),
        pltpu.VMEM_SHARED((HSIZE,), jnp.float32),
        pltpu.SemaphoreType.DMA,
        pltpu.SemaphoreType.DMA,
    ],
    compiler_params=pltpu.CompilerParams(needs_layout_passes=False),
)(_sc_hist_body)


def _tc_mm_body(c0_ref, c1_ref, e_ref, o_ref):
    o_ref[...] = jnp.dot(c0_ref[...] + c1_ref[...], e_ref[0],
                         preferred_element_type=jnp.float32)


def kernel(x, atom_to_cycle, emb_table):
    c = _sc_hist(atom_to_cycle, x)
    c2d = c.reshape(NC * STRIPE, 128)
    e4 = jnp.zeros((4, 128, HIDDEN), emb_table.dtype)
    for g in range(4):
        e4 = e4.at[g, g * 32:g * 32 + VOCAB, :].set(emb_table)
    out = pl.pallas_call(
        _tc_mm_body,
        grid=(4,),
        in_specs=[
            pl.BlockSpec((STRIPE, 128), lambda g: (0, 0)),
            pl.BlockSpec((STRIPE, 128), lambda g: (1, 0)),
            pl.BlockSpec((1, 128, HIDDEN), lambda g: (g, 0, 0)),
        ],
        out_specs=pl.BlockSpec((STRIPE, HIDDEN), lambda g: (g, 0)),
        out_shape=jax.ShapeDtypeStruct((N_CYCLES, HIDDEN), jnp.float32),
    )(c2d, c2d, e4)
    return out


# speedup vs baseline: 54.2810x; 1.1283x over previous
import functools

import jax
import jax.numpy as jnp
from jax import lax
from jax.experimental import pallas as pl
from jax.experimental.pallas import tpu as pltpu
from jax.experimental.pallas import tpu_sc as plsc

N_NODES = 10000
N_PAIRS = 320000
HIDDEN = 128
VOCAB = 22
N_CYCLES = 10000

STRIPE = 2560
HSIZE = STRIPE * 128
NC, NS = 2, 16
CHUNK = N_PAIRS // (NC * NS)
ROWS = CHUNK // 16
HTILE = HSIZE // NS
ZCH = 2048
SLAB = 10240


def _sc_hist_body(atc_hbm, x_hbm, c_hbm,
                  x_v, atc_v, idx_v, ones_v, zero_v, wb_v, hist_sh,
                  in_sem, z_sem):
    cid = lax.axis_index("c")
    s = lax.axis_index("s")
    w = cid * NS + s
    base = w * CHUNK

    start = jnp.minimum(base - base % 128, N_PAIRS - SLAB)
    start = pl.multiple_of(start, 128)
    off = base - start
    cp_x = pltpu.async_copy(x_hbm, x_v, in_sem)
    cp_a = pltpu.async_copy(atc_hbm.at[:, pl.ds(start, SLAB)], atc_v, in_sem)

    @plsc.parallel_loop(0, ZCH, step=16, unroll=8)
    def zloop(i):
        zero_v[pl.ds(i, 16)] = jnp.zeros((16,), jnp.float32)
    zcps = [
        pltpu.async_copy(zero_v, hist_sh.at[pl.ds(s * HTILE + j * ZCH, ZCH)],
                         z_sem)
        for j in range(HTILE // ZCH)
    ]

    ones16 = jnp.ones((16,), jnp.float32)
    @plsc.parallel_loop(0, CHUNK, step=16, unroll=8)
    def oloop(i):
        ones_v[pl.ds(i, 16)] = ones16

    cp_x.wait()
    cp_a.wait()

    @plsc.parallel_loop(0, CHUNK, step=16, unroll=4)
    def iloop(i):
        a0_16 = atc_v[0, pl.ds(off + i, 16)]
        v16 = plsc.load_gather(x_v, [a0_16])
        c16 = atc_v[1, pl.ds(off + i, 16)]
        g16 = (c16 * 26215) >> 26
        r16 = c16 - ((g16 << 11) + (g16 << 9))
        idx_v[pl.ds(i, 16)] = (r16 << 7) | (g16 << 5) | v16

    for cp in zcps:
        cp.wait()
    plsc.subcore_barrier()
    pltpu.sync_copy(ones_v, hist_sh.at[idx_v], add=True)
    plsc.subcore_barrier()

    pltpu.sync_copy(hist_sh.at[pl.ds(s * HTILE, HTILE)], wb_v)
    pltpu.sync_copy(wb_v, c_hbm.at[pl.ds(cid * HSIZE + s * HTILE, HTILE)])


_sc_hist = functools.partial(
    pl.kernel,
    out_type=jax.ShapeDtypeStruct((NC * HSIZE,), jnp.float32),
    mesh=plsc.VectorSubcoreMesh(core_axis_name="c", subcore_axis_name="s"),
    scratch_types=[
        pltpu.VMEM((N_NODES,), jnp.int32),
        pltpu.VMEM((2, SLAB), jnp.int32),
        pltpu.VMEM((CHUNK,), jnp.int32),
        pltpu.VMEM((CHUNK,), jnp.float32),
        pltpu.VMEM((ZCH,), jnp.float32),
        pltpu.VMEM((HTILE,), jnp.float32),
        pltpu.VMEM_SHARED((HSIZE,), jnp.float32),
        pltpu.SemaphoreType.DMA,
        pltpu.SemaphoreType.DMA,
    ],
    compiler_params=pltpu.CompilerParams(needs_layout_passes=False),
)(_sc_hist_body)


def _tc_mm_body(c0_ref, c1_ref, e_ref, o_ref):
    o_ref[...] = jnp.dot(c0_ref[...] + c1_ref[...], e_ref[0],
                         preferred_element_type=jnp.float32)


def kernel(x, atom_to_cycle, emb_table):
    c = _sc_hist(atom_to_cycle, x)
    c2d = c.reshape(NC * STRIPE, 128)
    e4 = jnp.zeros((4, 128, HIDDEN), emb_table.dtype)
    for g in range(4):
        e4 = e4.at[g, g * 32:g * 32 + VOCAB, :].set(emb_table)
    out = pl.pallas_call(
        _tc_mm_body,
        grid=(4,),
        in_specs=[
            pl.BlockSpec((STRIPE, 128), lambda g: (0, 0)),
            pl.BlockSpec((STRIPE, 128), lambda g: (1, 0)),
            pl.BlockSpec((1, 128, HIDDEN), lambda g: (g, 0, 0)),
        ],
        out_specs=pl.BlockSpec((STRIPE, HIDDEN), lambda g: (g, 0)),
        out_shape=jax.ShapeDtypeStruct((N_CYCLES, HIDDEN), jnp.float32),
    )(c2d, c2d, e4)
    return out
